# Initial kernel scaffold; baseline (speedup 1.0000x reference)
import jax
import jax.numpy as jnp
from jax.experimental import pallas as pl


def kernel(input_ids, weight):
    raise NotImplementedError("write your pallas kernel here")


# speedup vs baseline: 1.5696x; 1.5696x over previous
import functools

import jax
import jax.numpy as jnp
from jax import lax
from jax.experimental import pallas as pl
from jax.experimental.pallas import tpu as pltpu
from jax.experimental.pallas import tpu_sc as plsc

EMBEDDING_DIM = 32
BATCH, SEQ = 16384, 26
B_TOTAL = BATCH * SEQ
NUM_CORES, NUM_SUBCORES = 2, 16
NW = NUM_CORES * NUM_SUBCORES
B_PER_W = B_TOTAL // NW
CHUNK = 1664
NCHUNK = B_PER_W // CHUNK
NBUF = 2

_mesh = plsc.VectorSubcoreMesh(core_axis_name="c", subcore_axis_name="s")


@functools.partial(
    pl.kernel,
    out_type=jax.ShapeDtypeStruct((B_TOTAL, EMBEDDING_DIM), jnp.float32),
    mesh=_mesh,
    scratch_types=(
        [pltpu.VMEM((CHUNK,), jnp.int32) for _ in range(NBUF)]
        + [pltpu.VMEM((CHUNK, EMBEDDING_DIM), jnp.float32) for _ in range(NBUF)]
        + [pltpu.SemaphoreType.DMA for _ in range(NBUF)]
    ),
    compiler_params=pltpu.CompilerParams(use_tc_tiling_on_sc=False),
)
def _sc_gather(idx_hbm, table_hbm, out_hbm,
               idx0, idx1, rows0, rows1, sem0, sem1):
    idx_bufs = (idx0, idx1)
    row_bufs = (rows0, rows1)
    sems = (sem0, sem1)
    wid = lax.axis_index("s") * NUM_CORES + lax.axis_index("c")
    base = wid * B_PER_W

    def start(g, slot):
        off = base + g * CHUNK
        pltpu.sync_copy(idx_hbm.at[pl.ds(off, CHUNK)], idx_bufs[slot])
        return pltpu.async_copy(table_hbm.at[idx_bufs[slot]], row_bufs[slot],
                                sems[slot])

    inflight = [None] * NBUF
    inflight[0] = start(0, 0)
    for g in range(NCHUNK):
        slot = g % NBUF
        nxt = (g + 1) % NBUF
        if g + 1 < NCHUNK:
            inflight[nxt] = start(g + 1, nxt)
        inflight[slot].wait()
        pltpu.sync_copy(row_bufs[slot],
                        out_hbm.at[pl.ds(base + g * CHUNK, CHUNK)])


def kernel(input_ids, weight):
    flat = input_ids.reshape(-1).astype(jnp.int32)
    out = _sc_gather(flat, weight)
    return out.reshape(*input_ids.shape, EMBEDDING_DIM)
